# baseline (speedup 1.0000x reference)
import functools

import jax
import jax.numpy as jnp
from jax import lax
from jax.experimental import pallas as pl
from jax.experimental.pallas import tpu as pltpu
from jax.experimental.pallas import tpu_sc as plsc

_INFO = plsc.get_sparse_core_info()
_NC = _INFO.num_cores
_NS = _INFO.num_subcores
_NW = _NC * _NS

_CHUNK = 1024


@functools.lru_cache(maxsize=None)
def _make_gather(n_rows: int, embed: int):
    assert n_rows % _NW == 0
    per_w = n_rows // _NW
    assert per_w % _CHUNK == 0
    n_iters = per_w // _CHUNK

    mesh = plsc.VectorSubcoreMesh(core_axis_name="c", subcore_axis_name="s")

    @functools.partial(
        pl.kernel,
        mesh=mesh,
        out_type=jax.ShapeDtypeStruct((n_rows, embed), jnp.float32),
        scratch_types=[
            pltpu.VMEM((_CHUNK,), jnp.int32),
            pltpu.VMEM((_CHUNK, embed), jnp.float32),
            pltpu.SemaphoreType.DMA,
        ],
        compiler_params=pltpu.CompilerParams(use_tc_tiling_on_sc=False),
    )
    def gather_kernel(idx_hbm, table_hbm, out_hbm, idx_v, rows_v, sem):
        wid = lax.axis_index("s") * _NC + lax.axis_index("c")
        base = wid * per_w

        def body(g, carry):
            off = pl.multiple_of(base + g * _CHUNK, _CHUNK)
            pltpu.sync_copy(idx_hbm.at[pl.ds(off, _CHUNK)], idx_v)
            pltpu.async_copy(table_hbm.at[idx_v], rows_v, sem).wait()
            pltpu.sync_copy(rows_v, out_hbm.at[pl.ds(off, _CHUNK)])
            return carry

        lax.fori_loop(0, n_iters, body, 0, unroll=False)

    return gather_kernel


def kernel(token_id, table):
    b, s = token_id.shape
    v, d = table.shape
    flat = token_id.reshape(-1).astype(jnp.int32)
    out = _make_gather(b * s, d)(flat, table)
    return out.reshape(b, s, d)


# speedup vs baseline: 1.0147x; 1.0147x over previous
import functools

import jax
import jax.numpy as jnp
from jax import lax
from jax.experimental import pallas as pl
from jax.experimental.pallas import tpu as pltpu
from jax.experimental.pallas import tpu_sc as plsc

_INFO = plsc.get_sparse_core_info()
_NC = _INFO.num_cores
_NS = _INFO.num_subcores
_NW = _NC * _NS

_CHUNK = 800


@functools.lru_cache(maxsize=None)
def _make_gather(n_rows: int, embed: int):
    assert n_rows % _NW == 0
    per_w = n_rows // _NW
    assert per_w % _CHUNK == 0
    n_iters = per_w // _CHUNK

    mesh = plsc.VectorSubcoreMesh(core_axis_name="c", subcore_axis_name="s")

    @functools.partial(
        pl.kernel,
        mesh=mesh,
        out_type=jax.ShapeDtypeStruct((n_rows, embed), jnp.float32),
        scratch_types=[
            pltpu.VMEM((per_w,), jnp.int32),
            pltpu.VMEM((_CHUNK, embed), jnp.float32),
            pltpu.VMEM((_CHUNK, embed), jnp.float32),
            pltpu.SemaphoreType.DMA,
            pltpu.SemaphoreType.DMA,
            pltpu.SemaphoreType.DMA,
            pltpu.SemaphoreType.DMA,
        ],
        compiler_params=pltpu.CompilerParams(use_tc_tiling_on_sc=False),
    )
    def gather_kernel(idx_hbm, table_hbm, out_hbm, idx_v, rows_a, rows_b,
                      sem_ga, sem_gb, sem_sa, sem_sb):
        wid = lax.axis_index("s") * _NC + lax.axis_index("c")
        base = wid * per_w
        pltpu.sync_copy(idx_hbm.at[pl.ds(base, per_w)], idx_v)

        bufs = ((rows_a, sem_ga, sem_sa), (rows_b, sem_gb, sem_sb))

        def start_gather(g):
            rows, sem_g, _ = bufs[g % 2]
            return pltpu.async_copy(
                table_hbm.at[idx_v.at[pl.ds(g * _CHUNK, _CHUNK)]], rows, sem_g)

        def start_store(g):
            rows, _, sem_s = bufs[g % 2]
            return pltpu.async_copy(
                rows, out_hbm.at[pl.ds(base + g * _CHUNK, _CHUNK)], sem_s)

        gathers = {0: start_gather(0)}
        stores = {}
        for g in range(n_iters):
            gathers.pop(g).wait()
            if g > 0:
                stores.pop(g - 1).wait()
            if g + 1 < n_iters:
                gathers[g + 1] = start_gather(g + 1)
            stores[g] = start_store(g)
        stores.pop(n_iters - 1).wait()

    return gather_kernel


def kernel(token_id, table):
    b, s = token_id.shape
    v, d = table.shape
    flat = token_id.reshape(-1).astype(jnp.int32)
    out = _make_gather(b * s, d)(flat, table)
    return out.reshape(b, s, d)
